# baseline (speedup 1.0000x reference)
import functools
import math

import jax
import jax.numpy as jnp
from jax import lax
from jax.experimental import pallas as pl
from jax.experimental.pallas import tpu as pltpu
from jax.experimental.pallas import tpu_sc as plsc

_N = 1048576
_K = 64
_NC = 2
_NS = 16
_NW = _NC * _NS
_PER_W = _N // _NW
_NBUF = 8
_CHUNK = _PER_W // _NBUF
_PREFETCH = 2
_L = 16

_LN2 = math.log(2.0)

_mesh = plsc.VectorSubcoreMesh(core_axis_name="c", subcore_axis_name="s")


@functools.partial(
    pl.kernel,
    mesh=_mesh,
    compiler_params=pltpu.CompilerParams(needs_layout_passes=False),
    out_type=jax.ShapeDtypeStruct((_N,), jnp.float32),
    scratch_types=[
        pltpu.VMEM((_K,), jnp.float32),
        pltpu.VMEM((_K,), jnp.float32),
        pltpu.VMEM((_NBUF * _CHUNK,), jnp.int32),
        pltpu.VMEM((_NBUF * _CHUNK,), jnp.float32),
        pltpu.SemaphoreType.DMA,
        [pltpu.SemaphoreType.DMA] * _NBUF,
        [pltpu.SemaphoreType.DMA] * _NBUF,
    ],
)
def _gather_kernel(mp_hbm, z_hbm, out_hbm, mp_v, table_v, zb, ob,
                   sem_t, sem_i, sem_o):
    wid = lax.axis_index("s") * _NC + lax.axis_index("c")
    base = wid * _PER_W

    def _start_in(c):
        return pltpu.async_copy(
            z_hbm.at[pl.ds(base + c * _CHUNK, _CHUNK)],
            zb.at[pl.ds(c * _CHUNK, _CHUNK)],
            sem_i[c],
        )

    copies_in = [_start_in(c) for c in range(_PREFETCH)]
    t_copy = pltpu.async_copy(mp_hbm, mp_v, sem_t)
    t_copy.wait()

    for k in range(_K // _L):
        y = mp_v[pl.ds(k * _L, _L)]
        bits = lax.bitcast_convert_type(y, jnp.int32)
        w = bits.astype(jnp.float32) * (_LN2 / (1 << 23)) - (127.0 * _LN2)
        for _ in range(3):
            w = w + y * jnp.exp(-w) - 1.0
        table_v[pl.ds(k * _L, _L)] = w

    copies_out = []
    for c in range(_NBUF):
        if c + _PREFETCH < _NBUF:
            copies_in.append(_start_in(c + _PREFETCH))
        copies_in[c].wait()

        @plsc.parallel_loop(c * _CHUNK, (c + 1) * _CHUNK, _L, unroll=16)
        def _body(off):
            idx = zb[pl.ds(off, _L)]
            ob[pl.ds(off, _L)] = plsc.load_gather(table_v, [idx])

        copies_out.append(
            pltpu.async_copy(
                ob.at[pl.ds(c * _CHUNK, _CHUNK)],
                out_hbm.at[pl.ds(base + c * _CHUNK, _CHUNK)],
                sem_o[c],
            )
        )
    for c in range(_NBUF):
        copies_out[c].wait()


def kernel(z, x, mixture_probs):
    return _gather_kernel(mixture_probs, z.astype(jnp.int32))


# speedup vs baseline: 1.0101x; 1.0101x over previous
import functools
import math

import jax
import jax.numpy as jnp
from jax import lax
from jax.experimental import pallas as pl
from jax.experimental.pallas import tpu as pltpu
from jax.experimental.pallas import tpu_sc as plsc

_N = 1048576
_K = 64
_NC = 2
_NS = 16
_NW = _NC * _NS
_PER_W = _N // _NW
_SIZES = (2048, 8192, 8192, 8192, 4096, 2048)
_EDGES = tuple(sum(_SIZES[:i]) for i in range(len(_SIZES) + 1))
_L = 16

_LN2 = math.log(2.0)

_mesh = plsc.VectorSubcoreMesh(core_axis_name="c", subcore_axis_name="s")


@functools.partial(
    pl.kernel,
    mesh=_mesh,
    compiler_params=pltpu.CompilerParams(needs_layout_passes=False),
    out_type=jax.ShapeDtypeStruct((_N,), jnp.float32),
    scratch_types=[
        pltpu.VMEM((_K,), jnp.float32),
        pltpu.VMEM((_K,), jnp.float32),
        pltpu.VMEM((_PER_W,), jnp.int32),
        pltpu.VMEM((_PER_W,), jnp.float32),
        pltpu.SemaphoreType.DMA,
        [pltpu.SemaphoreType.DMA] * len(_SIZES),
        [pltpu.SemaphoreType.DMA] * len(_SIZES),
    ],
)
def _gather_kernel(mp_hbm, z_hbm, out_hbm, mp_v, table_v, zb, ob,
                   sem_t, sem_i, sem_o):
    wid = lax.axis_index("s") * _NC + lax.axis_index("c")
    base = wid * _PER_W

    t_copy = pltpu.async_copy(mp_hbm, mp_v, sem_t)
    copies_in = [
        pltpu.async_copy(
            z_hbm.at[pl.ds(base + _EDGES[c], _SIZES[c])],
            zb.at[pl.ds(_EDGES[c], _SIZES[c])],
            sem_i[c],
        )
        for c in range(len(_SIZES))
    ]
    t_copy.wait()

    for k in range(_K // _L):
        y = mp_v[pl.ds(k * _L, _L)]
        bits = lax.bitcast_convert_type(y, jnp.int32)
        w = bits.astype(jnp.float32) * (_LN2 / (1 << 23)) - (127.0 * _LN2)
        for _ in range(3):
            w = w + y * jnp.exp(-w) - 1.0
        table_v[pl.ds(k * _L, _L)] = w

    copies_out = []
    for c in range(len(_SIZES)):
        copies_in[c].wait()

        @plsc.parallel_loop(_EDGES[c], _EDGES[c + 1], _L, unroll=32)
        def _body(off):
            idx = zb[pl.ds(off, _L)]
            ob[pl.ds(off, _L)] = plsc.load_gather(table_v, [idx])

        copies_out.append(
            pltpu.async_copy(
                ob.at[pl.ds(_EDGES[c], _SIZES[c])],
                out_hbm.at[pl.ds(base + _EDGES[c], _SIZES[c])],
                sem_o[c],
            )
        )
    for cp in copies_out:
        cp.wait()


def kernel(z, x, mixture_probs):
    return _gather_kernel(mixture_probs, z.astype(jnp.int32))


# speedup vs baseline: 1.0287x; 1.0184x over previous
import functools
import math

import jax
import jax.numpy as jnp
from jax import lax
from jax.experimental import pallas as pl
from jax.experimental.pallas import tpu as pltpu
from jax.experimental.pallas import tpu_sc as plsc

_N = 1048576
_K = 64
_NC = 2
_NS = 16
_NW = _NC * _NS
_PER_W = _N // _NW
_SIZES = (4096, 12288, 12288, 4096)
_EDGES = tuple(sum(_SIZES[:i]) for i in range(len(_SIZES) + 1))
_L = 16

_LN2 = math.log(2.0)

_mesh = plsc.VectorSubcoreMesh(core_axis_name="c", subcore_axis_name="s")


@functools.partial(
    pl.kernel,
    mesh=_mesh,
    compiler_params=pltpu.CompilerParams(needs_layout_passes=False),
    out_type=jax.ShapeDtypeStruct((_N,), jnp.float32),
    scratch_types=[
        pltpu.VMEM((_K,), jnp.float32),
        pltpu.VMEM((_K,), jnp.float32),
        pltpu.VMEM((_PER_W,), jnp.int32),
        pltpu.VMEM((_PER_W,), jnp.float32),
        pltpu.SemaphoreType.DMA,
        [pltpu.SemaphoreType.DMA] * len(_SIZES),
        [pltpu.SemaphoreType.DMA] * len(_SIZES),
    ],
)
def _gather_kernel(mp_hbm, z_hbm, out_hbm, mp_v, table_v, zb, ob,
                   sem_t, sem_i, sem_o):
    wid = lax.axis_index("s") * _NC + lax.axis_index("c")
    base = wid * _PER_W

    t_copy = pltpu.async_copy(mp_hbm, mp_v, sem_t)
    copies_in = [
        pltpu.async_copy(
            z_hbm.at[pl.ds(base + _EDGES[c], _SIZES[c])],
            zb.at[pl.ds(_EDGES[c], _SIZES[c])],
            sem_i[c],
        )
        for c in range(len(_SIZES))
    ]
    t_copy.wait()

    for k in range(_K // _L):
        y = mp_v[pl.ds(k * _L, _L)]
        bits = lax.bitcast_convert_type(y, jnp.int32)
        w = bits.astype(jnp.float32) * (_LN2 / (1 << 23)) - (127.0 * _LN2)
        for _ in range(3):
            w = w + y * jnp.exp(-w) - 1.0
        table_v[pl.ds(k * _L, _L)] = w

    copies_out = []
    for c in range(len(_SIZES)):
        copies_in[c].wait()

        @plsc.parallel_loop(_EDGES[c], _EDGES[c + 1], _L, unroll=32)
        def _body(off):
            idx = zb[pl.ds(off, _L)]
            ob[pl.ds(off, _L)] = plsc.load_gather(table_v, [idx])

        copies_out.append(
            pltpu.async_copy(
                ob.at[pl.ds(_EDGES[c], _SIZES[c])],
                out_hbm.at[pl.ds(base + _EDGES[c], _SIZES[c])],
                sem_o[c],
            )
        )
    for cp in copies_out:
        cp.wait()


def kernel(z, x, mixture_probs):
    return _gather_kernel(mixture_probs, z.astype(jnp.int32))


# speedup vs baseline: 1.0413x; 1.0122x over previous
import functools
import math

import jax
import jax.numpy as jnp
from jax import lax
from jax.experimental import pallas as pl
from jax.experimental.pallas import tpu as pltpu
from jax.experimental.pallas import tpu_sc as plsc

_N = 1048576
_K = 64
_NC = 2
_NS = 16
_NW = _NC * _NS
_PER_W = _N // _NW
_SIZES = (8192, 8192, 8192, 8192)
_PREFETCH = 2
_EDGES = tuple(sum(_SIZES[:i]) for i in range(len(_SIZES) + 1))
_L = 16

_LN2 = math.log(2.0)

_mesh = plsc.VectorSubcoreMesh(core_axis_name="c", subcore_axis_name="s")


@functools.partial(
    pl.kernel,
    mesh=_mesh,
    compiler_params=pltpu.CompilerParams(needs_layout_passes=False),
    out_type=jax.ShapeDtypeStruct((_N,), jnp.float32),
    scratch_types=[
        pltpu.VMEM((_K,), jnp.float32),
        pltpu.VMEM((_K,), jnp.float32),
        pltpu.VMEM((_PER_W,), jnp.int32),
        pltpu.VMEM((_PER_W,), jnp.float32),
        pltpu.SemaphoreType.DMA,
        [pltpu.SemaphoreType.DMA] * len(_SIZES),
        [pltpu.SemaphoreType.DMA] * len(_SIZES),
    ],
)
def _gather_kernel(mp_hbm, z_hbm, out_hbm, mp_v, table_v, zb, ob,
                   sem_t, sem_i, sem_o):
    wid = lax.axis_index("s") * _NC + lax.axis_index("c")
    base = wid * _PER_W

    def _start_in(c):
        return pltpu.async_copy(
            z_hbm.at[pl.ds(base + _EDGES[c], _SIZES[c])],
            zb.at[pl.ds(_EDGES[c], _SIZES[c])],
            sem_i[c],
        )

    t_copy = pltpu.async_copy(mp_hbm, mp_v, sem_t)
    copies_in = [_start_in(c) for c in range(_PREFETCH)]
    t_copy.wait()

    for k in range(_K // _L):
        y = mp_v[pl.ds(k * _L, _L)]
        bits = lax.bitcast_convert_type(y, jnp.int32)
        w = bits.astype(jnp.float32) * (_LN2 / (1 << 23)) - (127.0 * _LN2)
        for _ in range(3):
            w = w + y * jnp.exp(-w) - 1.0
        table_v[pl.ds(k * _L, _L)] = w

    copies_out = []
    for c in range(len(_SIZES)):
        if c + _PREFETCH < len(_SIZES):
            copies_in.append(_start_in(c + _PREFETCH))
        copies_in[c].wait()

        @plsc.parallel_loop(_EDGES[c], _EDGES[c + 1], _L, unroll=32)
        def _body(off):
            idx = zb[pl.ds(off, _L)]
            ob[pl.ds(off, _L)] = plsc.load_gather(table_v, [idx])

        copies_out.append(
            pltpu.async_copy(
                ob.at[pl.ds(_EDGES[c], _SIZES[c])],
                out_hbm.at[pl.ds(base + _EDGES[c], _SIZES[c])],
                sem_o[c],
            )
        )
    for cp in copies_out:
        cp.wait()


def kernel(z, x, mixture_probs):
    return _gather_kernel(mixture_probs, z.astype(jnp.int32))
